# Initial kernel scaffold; baseline (speedup 1.0000x reference)
import jax
import jax.numpy as jnp
from jax.experimental import pallas as pl


def kernel(x, edge_src0, edge_dst0, etype0, edge_src1, edge_dst1, etype1, n_dst0, n_dst1, Wself, Wneigh, bconv, Wskip, bskip, gamma, beta, W1, b1, g1, be1, W2, b2):
    raise NotImplementedError("write your pallas kernel here")


# speedup vs baseline: 16.4931x; 16.4931x over previous
import functools

import jax
import jax.numpy as jnp
from jax import lax
from jax.experimental import pallas as pl
from jax.experimental.pallas import tpu as pltpu
from jax.experimental.pallas import tpu_sc as plsc

NE = 5
D = 128
LN = 16
NC, NS = 2, 16
NW = NC * NS
EPS = 1e-5
B = 80

N0, N1 = 4000, 1024
N0P = 4096
CNT0 = 20480
CNT1 = N1 * NE


def _sc_mesh():
    return plsc.VectorSubcoreMesh(core_axis_name="c", subcore_axis_name="s",
                                  num_cores=NC, num_subcores=NS)


def _proj_body(x_ref, wa_ref, wb_ref, bb_ref, oa_ref, ob_ref):
    xv = x_ref[...]
    oa_ref[...] = jnp.dot(xv, wa_ref[...], preferred_element_type=jnp.float32, precision=lax.Precision.HIGHEST)
    ob_ref[...] = (jnp.dot(xv, wb_ref[...], preferred_element_type=jnp.float32, precision=lax.Precision.HIGHEST)
                   + bb_ref[...])


def _tc_project(x, wa, wb, bb, br):
    m, k = x.shape
    na, nb = wa.shape[1], wb.shape[1]
    return pl.pallas_call(
        _proj_body,
        grid=(m // br,),
        in_specs=[pl.BlockSpec((br, k), lambda i: (i, 0)),
                  pl.BlockSpec((k, na), lambda i: (0, 0)),
                  pl.BlockSpec((k, nb), lambda i: (0, 0)),
                  pl.BlockSpec((1, nb), lambda i: (0, 0))],
        out_specs=[pl.BlockSpec((br, na), lambda i: (i, 0)),
                   pl.BlockSpec((br, nb), lambda i: (i, 0))],
        out_shape=[jax.ShapeDtypeStruct((m, na), jnp.float32),
                   jax.ShapeDtypeStruct((m, nb), jnp.float32)],
    )(x, wa, wb, bb.reshape(1, nb))


def _bn_elu(h, g, b):
    mu = jnp.mean(h, axis=0, keepdims=True)
    var = jnp.mean((h - mu) ** 2, axis=0, keepdims=True)
    h = (h - mu) / jnp.sqrt(var + EPS) * g + b
    return jnp.where(h > 0, h, jnp.exp(h) - 1.0)


def _mid_body(hd_ref, hp_ref, g_ref, b_ref, wa_ref, wb_ref, bb_ref,
              oa_ref, ob_ref):
    h = hd_ref[...] + hp_ref[0] + hp_ref[1]
    h = _bn_elu(h, g_ref[...], b_ref[...])
    oa_ref[...] = jnp.dot(h, wa_ref[...], preferred_element_type=jnp.float32, precision=lax.Precision.HIGHEST)
    ob_ref[...] = (jnp.dot(h, wb_ref[...], preferred_element_type=jnp.float32, precision=lax.Precision.HIGHEST)
                   + bb_ref[...])


def _tc_mid(hd, hp, g, b, wa, wb, bb, n):
    na, nb = wa.shape[1], wb.shape[1]
    return pl.pallas_call(
        _mid_body,
        grid=(1,),
        in_specs=[pl.BlockSpec((n, D), lambda i: (0, 0)),
                  pl.BlockSpec((NC, n, D), lambda i: (0, 0, 0)),
                  pl.BlockSpec((1, D), lambda i: (0, 0)),
                  pl.BlockSpec((1, D), lambda i: (0, 0)),
                  pl.BlockSpec((D, na), lambda i: (0, 0)),
                  pl.BlockSpec((D, nb), lambda i: (0, 0)),
                  pl.BlockSpec((1, nb), lambda i: (0, 0))],
        out_specs=[pl.BlockSpec((n, na), lambda i: (i, 0)),
                   pl.BlockSpec((n, nb), lambda i: (i, 0))],
        out_shape=[jax.ShapeDtypeStruct((n, na), jnp.float32),
                   jax.ShapeDtypeStruct((n, nb), jnp.float32)],
    )(hd, hp, g.reshape(1, D), b.reshape(1, D), wa, wb, bb.reshape(1, nb))


def _fin_body(hd_ref, hp_ref, g_ref, b_ref, w1_ref, b1_ref, g1_ref, be1_ref,
              w2_ref, b2_ref, o_ref):
    h = hd_ref[...] + hp_ref[0] + hp_ref[1]
    h = _bn_elu(h, g_ref[...], b_ref[...])
    z = jnp.dot(h, w1_ref[...], preferred_element_type=jnp.float32, precision=lax.Precision.HIGHEST) + b1_ref[...]
    mu = jnp.mean(z, axis=0, keepdims=True)
    var = jnp.mean((z - mu) ** 2, axis=0, keepdims=True)
    z = (z - mu) / jnp.sqrt(var + EPS) * g1_ref[...] + be1_ref[...]
    z = jnp.maximum(z, 0.0)
    o_ref[...] = (jnp.dot(z, w2_ref[...], preferred_element_type=jnp.float32, precision=lax.Precision.HIGHEST)
                  + b2_ref[...])


def _tc_final(hd, hp, g, b, w1, b1, g1, be1, w2p, b2p):
    n = hp.shape[1]
    no = w2p.shape[1]
    return pl.pallas_call(
        _fin_body,
        grid=(1,),
        in_specs=[pl.BlockSpec((n, D), lambda i: (0, 0)),
                  pl.BlockSpec((NC, n, D), lambda i: (0, 0, 0)),
                  pl.BlockSpec((1, D), lambda i: (0, 0)),
                  pl.BlockSpec((1, D), lambda i: (0, 0)),
                  pl.BlockSpec((D, D), lambda i: (0, 0)),
                  pl.BlockSpec((1, D), lambda i: (0, 0)),
                  pl.BlockSpec((1, D), lambda i: (0, 0)),
                  pl.BlockSpec((1, D), lambda i: (0, 0)),
                  pl.BlockSpec((D, no), lambda i: (0, 0)),
                  pl.BlockSpec((1, no), lambda i: (0, 0))],
        out_specs=pl.BlockSpec((n, no), lambda i: (0, 0)),
        out_shape=jax.ShapeDtypeStruct((n, no), jnp.float32),
    )(hd, hp, g.reshape(1, D), b.reshape(1, D), w1, b1.reshape(1, D),
      g1.reshape(1, D), be1.reshape(1, D), w2p, b2p.reshape(1, no))


def _sc_counts(d0, e0, d1, e1):
    ew0 = d0.shape[0] // NS
    ew1 = d1.shape[0] // NS

    @functools.partial(
        pl.kernel, mesh=_sc_mesh(),
        compiler_params=pltpu.CompilerParams(needs_layout_passes=False),
        out_type=[jax.ShapeDtypeStruct((CNT0,), jnp.float32),
                  jax.ShapeDtypeStruct((CNT1,), jnp.float32)],
        scratch_types=[
            pltpu.VMEM((ew0,), jnp.int32),
            pltpu.VMEM((ew0,), jnp.int32),
            pltpu.VMEM((1, B), jnp.int32),
            pltpu.VMEM((B,), jnp.float32),
            pltpu.VMEM((CNT0 // NW,), jnp.float32),
            pltpu.VMEM_SHARED((CNT0,), jnp.float32),
            pltpu.VMEM_SHARED((CNT1,), jnp.float32),
        ],
    )
    def k(d0h, e0h, d1h, e1h, r0h, r1h, dbuf, ebuf, kbuf, ones, rbuf,
          c0sh, c1sh):
        cid = lax.axis_index("c")
        sid = lax.axis_index("s")
        wid = cid * NS + sid
        for i in range(B // LN):
            ones[pl.ds(i * LN, LN)] = jnp.full((LN,), 1.0, jnp.float32)

        def zb(i, c):
            rbuf[pl.ds(i * LN, LN)] = jnp.zeros((LN,), jnp.float32)
            return c
        lax.fori_loop(0, (CNT0 // NW) // LN, zb, 0)
        s0 = CNT0 // NS
        pltpu.sync_copy(rbuf, c0sh.at[pl.ds(sid * s0, CNT0 // NW)])
        pltpu.sync_copy(rbuf, c0sh.at[pl.ds(sid * s0 + CNT0 // NW, CNT0 // NW)])
        pltpu.sync_copy(rbuf.at[pl.ds(0, CNT1 // NS)],
                        c1sh.at[pl.ds(sid * (CNT1 // NS), CNT1 // NS)])
        plsc.subcore_barrier()

        pltpu.sync_copy(d0h.at[pl.ds(sid * ew0, ew0)], dbuf)
        pltpu.sync_copy(e0h.at[pl.ds(sid * ew0, ew0)], ebuf)

        def b0(b, c):
            for i in range(B // LN):
                d16 = dbuf[pl.ds(b * B + i * LN, LN)]
                e16 = ebuf[pl.ds(b * B + i * LN, LN)]
                kbuf[0, pl.ds(i * LN, LN)] = d16 * NE + e16
            pltpu.sync_copy(ones, c0sh.at[kbuf.at[0]], add=True)
            return c
        lax.fori_loop(0, ew0 // B, b0, 0)

        pltpu.sync_copy(d1h.at[pl.ds(sid * ew1, ew1)], dbuf.at[pl.ds(0, ew1)])
        pltpu.sync_copy(e1h.at[pl.ds(sid * ew1, ew1)], ebuf.at[pl.ds(0, ew1)])

        def b1(b, c):
            for i in range(B // LN):
                d16 = dbuf[pl.ds(b * B + i * LN, LN)]
                e16 = ebuf[pl.ds(b * B + i * LN, LN)]
                kbuf[0, pl.ds(i * LN, LN)] = d16 * NE + e16
            pltpu.sync_copy(ones, c1sh.at[kbuf.at[0]], add=True)
            return c
        lax.fori_loop(0, ew1 // B, b1, 0)
        plsc.subcore_barrier()

        w0 = CNT0 // NW
        pltpu.sync_copy(c0sh.at[pl.ds(wid * w0, w0)], rbuf)

        def rp(i, c):
            v = rbuf[pl.ds(i * LN, LN)]
            rbuf[pl.ds(i * LN, LN)] = 1.0 / jnp.maximum(v, 1.0)
            return c
        lax.fori_loop(0, w0 // LN, rp, 0)
        pltpu.sync_copy(rbuf, r0h.at[pl.ds(wid * w0, w0)])

        w1 = CNT1 // NW
        pltpu.sync_copy(c1sh.at[pl.ds(wid * w1, w1)], rbuf.at[pl.ds(0, w1)])

        def rp1(i, c):
            v = rbuf[pl.ds(i * LN, LN)]
            rbuf[pl.ds(i * LN, LN)] = 1.0 / jnp.maximum(v, 1.0)
            return c
        lax.fori_loop(0, w1 // LN, rp1, 0)
        pltpu.sync_copy(rbuf.at[pl.ds(0, w1)], r1h.at[pl.ds(wid * w1, w1)])

    return k(d0, e0, d1, e1)


def _sc_agg(xn, src, dst, et, recip, n_pad):
    epw = src.shape[0] // NW
    nb = epw // B
    cnt = recip.shape[0]
    rps = n_pad // NS

    @functools.partial(
        pl.kernel, mesh=_sc_mesh(),
        compiler_params=pltpu.CompilerParams(needs_layout_passes=False),
        out_type=jax.ShapeDtypeStruct((NC, n_pad, D), jnp.float32),
        scratch_types=[
            pltpu.VMEM((epw,), jnp.int32),
            pltpu.VMEM((epw,), jnp.int32),
            pltpu.VMEM((epw,), jnp.int32),
            pltpu.VMEM((cnt,), jnp.float32),
            pltpu.VMEM((B,), jnp.int32),
            pltpu.VMEM((1, B), jnp.int32),
            pltpu.VMEM((B,), jnp.float32),
            pltpu.VMEM((B, D), jnp.float32),
            pltpu.VMEM_SHARED((n_pad, D), jnp.float32),
            pltpu.SemaphoreType.DMA,
        ],
    )
    def k(xn_h, src_h, dst_h, et_h, rc_h, out_h,
          sbuf, dbuf, ebuf, rcl, gix, dix, scl, rows, acc, sem):
        cid = lax.axis_index("c")
        sid = lax.axis_index("s")
        wid = cid * NS + sid
        pltpu.sync_copy(src_h.at[pl.ds(wid * epw, epw)], sbuf)
        pltpu.sync_copy(dst_h.at[pl.ds(wid * epw, epw)], dbuf)
        pltpu.sync_copy(et_h.at[pl.ds(wid * epw, epw)], ebuf)
        pltpu.sync_copy(rc_h, rcl)

        def zr(i, c):
            for kb in range(D // LN):
                rows[i, pl.ds(kb * LN, LN)] = jnp.zeros((LN,), jnp.float32)
            return c
        lax.fori_loop(0, B, zr, 0)
        off = 0
        while off < rps:
            n = min(B, rps - off)
            pltpu.sync_copy(rows.at[pl.ds(0, n)],
                            acc.at[pl.ds(sid * rps + off, n)])
            off += n
        plsc.subcore_barrier()

        def bat(b, c):
            for i in range(B // LN):
                s16 = sbuf[pl.ds(b * B + i * LN, LN)]
                e16 = ebuf[pl.ds(b * B + i * LN, LN)]
                d16 = dbuf[pl.ds(b * B + i * LN, LN)]
                gix[pl.ds(i * LN, LN)] = s16 * NE + e16
                dix[0, pl.ds(i * LN, LN)] = d16
                scl[pl.ds(i * LN, LN)] = plsc.load_gather(rcl, [d16 * NE + e16])
            pltpu.async_copy(xn_h.at[gix], rows, sem).wait()

            def se(g, c2):
                sv16 = scl[pl.ds(g * LN, LN)]
                base = g * LN
                for j in range(LN):
                    sv = sv16[j]
                    for kb in range(D // LN):
                        rows[base + j, pl.ds(kb * LN, LN)] = (
                            rows[base + j, pl.ds(kb * LN, LN)] * sv)
                return c2
            lax.fori_loop(0, B // LN, se, 0)
            pltpu.sync_copy(rows, acc.at[dix.at[0]], add=True)
            return c
        lax.fori_loop(0, nb, bat, 0)
        plsc.subcore_barrier()

        off = 0
        while off < rps:
            n = min(B, rps - off)
            pltpu.sync_copy(acc.at[pl.ds(sid * rps + off, n)],
                            out_h.at[cid, pl.ds(sid * rps + off, n)])
            off += n

    return k(xn, src, dst, et, recip)


def kernel(x, edge_src0, edge_dst0, etype0, edge_src1, edge_dst1, etype1,
           n_dst0, n_dst1, Wself, Wneigh, bconv, Wskip, bskip, gamma, beta,
           W1, b1, g1, be1, W2, b2):
    wn0 = jnp.transpose(Wneigh[0], (1, 0, 2)).reshape(D, NE * D)
    wc0 = Wskip[0] + Wself[0].sum(0)
    bc0 = bskip[0] + bconv[0].sum(0)
    wn1 = jnp.transpose(Wneigh[1], (1, 0, 2)).reshape(D, NE * D)
    wc1 = Wskip[1] + Wself[1].sum(0)
    bc1 = bskip[1] + bconv[1].sum(0)
    dout = W2.shape[1]
    w2p = jnp.pad(W2, ((0, 0), (0, 256 - dout)))
    b2p = jnp.pad(b2, (0, 256 - dout))

    recip0, recip1 = _sc_counts(edge_dst0, etype0, edge_dst1, etype1)

    y0a, y0b = _tc_project(x, wn0, wc0, bc0, br=1000)
    xn0 = y0a.reshape(x.shape[0] * NE, D)
    hp0 = _sc_agg(xn0, edge_src0, edge_dst0, etype0, recip0, N0P)

    y1a, y1b = _tc_mid(y0b, hp0, gamma[0], beta[0], wn1, wc1, bc1, N0)
    xn1 = y1a.reshape(N0 * NE, D)
    hp1 = _sc_agg(xn1, edge_src1, edge_dst1, etype1, recip1, N1)

    out = _tc_final(y1b, hp1, gamma[1], beta[1], W1, b1, g1, be1, w2p, b2p)
    return out[:, :dout]


# speedup vs baseline: 25.4816x; 1.5450x over previous
import functools

import jax
import jax.numpy as jnp
from jax import lax
from jax.experimental import pallas as pl
from jax.experimental.pallas import tpu as pltpu
from jax.experimental.pallas import tpu_sc as plsc

NE = 5
D = 128
LN = 16
NC, NS = 2, 16
NW = NC * NS
EPS = 1e-5
B = 80

N0, N1 = 4000, 1024
N0P = 4096
CNT0 = 20480
CNT1 = N1 * NE


def _sc_mesh():
    return plsc.VectorSubcoreMesh(core_axis_name="c", subcore_axis_name="s",
                                  num_cores=NC, num_subcores=NS)


def _proj_body(x_ref, wa_ref, wb_ref, bb_ref, oa_ref, ob_ref):
    xv = x_ref[...]
    oa_ref[...] = jnp.dot(xv, wa_ref[...], preferred_element_type=jnp.float32, precision=lax.Precision.HIGHEST)
    ob_ref[...] = (jnp.dot(xv, wb_ref[...], preferred_element_type=jnp.float32, precision=lax.Precision.HIGHEST)
                   + bb_ref[...])


def _tc_project(x, wa, wb, bb, br):
    m, k = x.shape
    na, nb = wa.shape[1], wb.shape[1]
    return pl.pallas_call(
        _proj_body,
        grid=(m // br,),
        in_specs=[pl.BlockSpec((br, k), lambda i: (i, 0)),
                  pl.BlockSpec((k, na), lambda i: (0, 0)),
                  pl.BlockSpec((k, nb), lambda i: (0, 0)),
                  pl.BlockSpec((1, nb), lambda i: (0, 0))],
        out_specs=[pl.BlockSpec((br, na), lambda i: (i, 0)),
                   pl.BlockSpec((br, nb), lambda i: (i, 0))],
        out_shape=[jax.ShapeDtypeStruct((m, na), jnp.float32),
                   jax.ShapeDtypeStruct((m, nb), jnp.float32)],
    )(x, wa, wb, bb.reshape(1, nb))


def _bn_elu(h, g, b):
    mu = jnp.mean(h, axis=0, keepdims=True)
    var = jnp.mean((h - mu) ** 2, axis=0, keepdims=True)
    h = (h - mu) / jnp.sqrt(var + EPS) * g + b
    return jnp.where(h > 0, h, jnp.exp(h) - 1.0)


def _mid_body(hd_ref, hp_ref, g_ref, b_ref, wa_ref, wb_ref, bb_ref,
              oa_ref, ob_ref):
    h = hd_ref[...] + hp_ref[0] + hp_ref[1]
    h = _bn_elu(h, g_ref[...], b_ref[...])
    oa_ref[...] = jnp.dot(h, wa_ref[...], preferred_element_type=jnp.float32, precision=lax.Precision.HIGHEST)
    ob_ref[...] = (jnp.dot(h, wb_ref[...], preferred_element_type=jnp.float32, precision=lax.Precision.HIGHEST)
                   + bb_ref[...])


def _tc_mid(hd, hp, g, b, wa, wb, bb, n):
    na, nb = wa.shape[1], wb.shape[1]
    return pl.pallas_call(
        _mid_body,
        grid=(1,),
        in_specs=[pl.BlockSpec((n, D), lambda i: (0, 0)),
                  pl.BlockSpec((NC, n, D), lambda i: (0, 0, 0)),
                  pl.BlockSpec((1, D), lambda i: (0, 0)),
                  pl.BlockSpec((1, D), lambda i: (0, 0)),
                  pl.BlockSpec((D, na), lambda i: (0, 0)),
                  pl.BlockSpec((D, nb), lambda i: (0, 0)),
                  pl.BlockSpec((1, nb), lambda i: (0, 0))],
        out_specs=[pl.BlockSpec((n, na), lambda i: (i, 0)),
                   pl.BlockSpec((n, nb), lambda i: (i, 0))],
        out_shape=[jax.ShapeDtypeStruct((n, na), jnp.float32),
                   jax.ShapeDtypeStruct((n, nb), jnp.float32)],
    )(hd, hp, g.reshape(1, D), b.reshape(1, D), wa, wb, bb.reshape(1, nb))


def _fin_body(hd_ref, hp_ref, g_ref, b_ref, w1_ref, b1_ref, g1_ref, be1_ref,
              w2_ref, b2_ref, o_ref):
    h = hd_ref[...] + hp_ref[0] + hp_ref[1]
    h = _bn_elu(h, g_ref[...], b_ref[...])
    z = jnp.dot(h, w1_ref[...], preferred_element_type=jnp.float32, precision=lax.Precision.HIGHEST) + b1_ref[...]
    mu = jnp.mean(z, axis=0, keepdims=True)
    var = jnp.mean((z - mu) ** 2, axis=0, keepdims=True)
    z = (z - mu) / jnp.sqrt(var + EPS) * g1_ref[...] + be1_ref[...]
    z = jnp.maximum(z, 0.0)
    o_ref[...] = (jnp.dot(z, w2_ref[...], preferred_element_type=jnp.float32, precision=lax.Precision.HIGHEST)
                  + b2_ref[...])


def _tc_final(hd, hp, g, b, w1, b1, g1, be1, w2p, b2p):
    n = hp.shape[1]
    no = w2p.shape[1]
    return pl.pallas_call(
        _fin_body,
        grid=(1,),
        in_specs=[pl.BlockSpec((n, D), lambda i: (0, 0)),
                  pl.BlockSpec((NC, n, D), lambda i: (0, 0, 0)),
                  pl.BlockSpec((1, D), lambda i: (0, 0)),
                  pl.BlockSpec((1, D), lambda i: (0, 0)),
                  pl.BlockSpec((D, D), lambda i: (0, 0)),
                  pl.BlockSpec((1, D), lambda i: (0, 0)),
                  pl.BlockSpec((1, D), lambda i: (0, 0)),
                  pl.BlockSpec((1, D), lambda i: (0, 0)),
                  pl.BlockSpec((D, no), lambda i: (0, 0)),
                  pl.BlockSpec((1, no), lambda i: (0, 0))],
        out_specs=pl.BlockSpec((n, no), lambda i: (0, 0)),
        out_shape=jax.ShapeDtypeStruct((n, no), jnp.float32),
    )(hd, hp, g.reshape(1, D), b.reshape(1, D), w1, b1.reshape(1, D),
      g1.reshape(1, D), be1.reshape(1, D), w2p, b2p.reshape(1, no))


def _sc_counts(d0, e0, d1, e1):
    ew0 = d0.shape[0] // NS
    ew1 = d1.shape[0] // NS

    @functools.partial(
        pl.kernel, mesh=_sc_mesh(),
        compiler_params=pltpu.CompilerParams(needs_layout_passes=False),
        out_type=[jax.ShapeDtypeStruct((CNT0,), jnp.float32),
                  jax.ShapeDtypeStruct((CNT1,), jnp.float32)],
        scratch_types=[
            pltpu.VMEM((ew0,), jnp.int32),
            pltpu.VMEM((ew0,), jnp.int32),
            pltpu.VMEM((1, B), jnp.int32),
            pltpu.VMEM((B,), jnp.float32),
            pltpu.VMEM((CNT0 // NW,), jnp.float32),
            pltpu.VMEM_SHARED((CNT0,), jnp.float32),
            pltpu.VMEM_SHARED((CNT1,), jnp.float32),
        ],
    )
    def k(d0h, e0h, d1h, e1h, r0h, r1h, dbuf, ebuf, kbuf, ones, rbuf,
          c0sh, c1sh):
        cid = lax.axis_index("c")
        sid = lax.axis_index("s")
        wid = cid * NS + sid
        for i in range(B // LN):
            ones[pl.ds(i * LN, LN)] = jnp.full((LN,), 1.0, jnp.float32)

        def zb(i, c):
            rbuf[pl.ds(i * LN, LN)] = jnp.zeros((LN,), jnp.float32)
            return c
        lax.fori_loop(0, (CNT0 // NW) // LN, zb, 0)
        s0 = CNT0 // NS
        pltpu.sync_copy(rbuf, c0sh.at[pl.ds(sid * s0, CNT0 // NW)])
        pltpu.sync_copy(rbuf, c0sh.at[pl.ds(sid * s0 + CNT0 // NW, CNT0 // NW)])
        pltpu.sync_copy(rbuf.at[pl.ds(0, CNT1 // NS)],
                        c1sh.at[pl.ds(sid * (CNT1 // NS), CNT1 // NS)])
        plsc.subcore_barrier()

        pltpu.sync_copy(d0h.at[pl.ds(sid * ew0, ew0)], dbuf)
        pltpu.sync_copy(e0h.at[pl.ds(sid * ew0, ew0)], ebuf)

        def b0(b, c):
            for i in range(B // LN):
                d16 = dbuf[pl.ds(b * B + i * LN, LN)]
                e16 = ebuf[pl.ds(b * B + i * LN, LN)]
                kbuf[0, pl.ds(i * LN, LN)] = d16 * NE + e16
            pltpu.sync_copy(ones, c0sh.at[kbuf.at[0]], add=True)
            return c
        lax.fori_loop(0, ew0 // B, b0, 0)

        pltpu.sync_copy(d1h.at[pl.ds(sid * ew1, ew1)], dbuf.at[pl.ds(0, ew1)])
        pltpu.sync_copy(e1h.at[pl.ds(sid * ew1, ew1)], ebuf.at[pl.ds(0, ew1)])

        def b1(b, c):
            for i in range(B // LN):
                d16 = dbuf[pl.ds(b * B + i * LN, LN)]
                e16 = ebuf[pl.ds(b * B + i * LN, LN)]
                kbuf[0, pl.ds(i * LN, LN)] = d16 * NE + e16
            pltpu.sync_copy(ones, c1sh.at[kbuf.at[0]], add=True)
            return c
        lax.fori_loop(0, ew1 // B, b1, 0)
        plsc.subcore_barrier()

        w0 = CNT0 // NW
        pltpu.sync_copy(c0sh.at[pl.ds(wid * w0, w0)], rbuf)

        def rp(i, c):
            v = rbuf[pl.ds(i * LN, LN)]
            rbuf[pl.ds(i * LN, LN)] = 1.0 / jnp.maximum(v, 1.0)
            return c
        lax.fori_loop(0, w0 // LN, rp, 0)
        pltpu.sync_copy(rbuf, r0h.at[pl.ds(wid * w0, w0)])

        w1 = CNT1 // NW
        pltpu.sync_copy(c1sh.at[pl.ds(wid * w1, w1)], rbuf.at[pl.ds(0, w1)])

        def rp1(i, c):
            v = rbuf[pl.ds(i * LN, LN)]
            rbuf[pl.ds(i * LN, LN)] = 1.0 / jnp.maximum(v, 1.0)
            return c
        lax.fori_loop(0, w1 // LN, rp1, 0)
        pltpu.sync_copy(rbuf.at[pl.ds(0, w1)], r1h.at[pl.ds(wid * w1, w1)])

    return k(d0, e0, d1, e1)


def _sc_agg(xn, src, dst, et, recip, n_pad):
    epw = src.shape[0] // NW
    nb = epw // B
    cnt = recip.shape[0]
    rps = n_pad // NS
    nbuf = 4
    jmax = (nb - 2) // nbuf
    assert jmax >= 2 and nb - nbuf * jmax >= 2

    @functools.partial(
        pl.kernel, mesh=_sc_mesh(),
        compiler_params=pltpu.CompilerParams(needs_layout_passes=False),
        out_type=jax.ShapeDtypeStruct((NC, n_pad, D), jnp.float32),
        scratch_types=[
            pltpu.VMEM((epw,), jnp.int32),
            pltpu.VMEM((epw,), jnp.int32),
            pltpu.VMEM((epw,), jnp.int32),
            pltpu.VMEM((cnt,), jnp.float32),
            pltpu.VMEM((nbuf, B), jnp.int32),
            pltpu.VMEM((nbuf, B), jnp.int32),
            pltpu.VMEM((nbuf, B), jnp.float32),
            pltpu.VMEM((nbuf, B, D), jnp.float32),
            pltpu.VMEM_SHARED((n_pad, D), jnp.float32),
            pltpu.SemaphoreType.DMA,
            pltpu.SemaphoreType.DMA,
            pltpu.SemaphoreType.DMA,
            pltpu.SemaphoreType.DMA,
            pltpu.SemaphoreType.DMA,
            pltpu.SemaphoreType.DMA,
            pltpu.SemaphoreType.DMA,
            pltpu.SemaphoreType.DMA,
        ],
    )
    def k(xn_h, src_h, dst_h, et_h, rc_h, out_h,
          sbuf, dbuf, ebuf, rcl, gix, dix, scl, rows, acc,
          gs0, gs1, gs2, gs3, ss0, ss1, ss2, ss3):
        gsem = (gs0, gs1, gs2, gs3)
        ssem = (ss0, ss1, ss2, ss3)
        cid = lax.axis_index("c")
        sid = lax.axis_index("s")
        wid = cid * NS + sid
        pltpu.sync_copy(src_h.at[pl.ds(wid * epw, epw)], sbuf)
        pltpu.sync_copy(dst_h.at[pl.ds(wid * epw, epw)], dbuf)
        pltpu.sync_copy(et_h.at[pl.ds(wid * epw, epw)], ebuf)
        pltpu.sync_copy(rc_h, rcl)

        def zr(i, c):
            for kb in range(D // LN):
                rows[0, i, pl.ds(kb * LN, LN)] = jnp.zeros((LN,), jnp.float32)
            return c
        lax.fori_loop(0, B, zr, 0)
        off = 0
        while off < rps:
            n = min(B, rps - off)
            pltpu.sync_copy(rows.at[0, pl.ds(0, n)],
                            acc.at[pl.ds(sid * rps + off, n)])
            off += n
        plsc.subcore_barrier()

        def prep(b, p):
            for i in range(B // LN):
                s16 = sbuf[pl.ds(b * B + i * LN, LN)]
                e16 = ebuf[pl.ds(b * B + i * LN, LN)]
                d16 = dbuf[pl.ds(b * B + i * LN, LN)]
                gix[p, pl.ds(i * LN, LN)] = s16 * NE + e16
                dix[p, pl.ds(i * LN, LN)] = d16
                scl[p, pl.ds(i * LN, LN)] = plsc.load_gather(rcl, [d16 * NE + e16])

        def issue(b, p, first=False):
            if not first:
                pltpu.make_async_copy(rows.at[p], acc.at[dix.at[p]],
                                      ssem[p]).wait()
            prep(b, p)
            pltpu.async_copy(xn_h.at[gix.at[p]], rows.at[p], gsem[p])

        def complete(p):
            pltpu.make_async_copy(xn_h.at[gix.at[p]], rows.at[p],
                                  gsem[p]).wait()

            def se(g, c2):
                sv16 = scl[p, pl.ds(g * LN, LN)]
                for j in range(LN):
                    sv = sv16[j]
                    for kb in range(D // LN):
                        rows[p, g * LN + j, pl.ds(kb * LN, LN)] = (
                            rows[p, g * LN + j, pl.ds(kb * LN, LN)] * sv)
                return c2
            lax.fori_loop(0, B // LN, se, 0)
            pltpu.async_copy(rows.at[p], acc.at[dix.at[p]], ssem[p], add=True)

        issue(0, 0, first=True)
        issue(1, 1, first=True)
        complete(0)
        issue(2, 2, first=True)
        complete(1)
        issue(3, 3, first=True)
        complete(2)
        issue(4, 0)
        complete(3)
        issue(5, 1)

        def main(j, c):
            for kq in range(nbuf):
                complete(kq)
                issue(nbuf * j + kq + 2, (kq + 2) % nbuf)
            return c
        lax.fori_loop(1, jmax, main, 0)

        for b in range(nbuf * jmax, nb):
            complete(b % nbuf)
            if b + 2 < nb:
                issue(b + 2, (b + 2) % nbuf)
        for p in range(nbuf):
            pltpu.make_async_copy(rows.at[p], acc.at[dix.at[p]],
                                  ssem[p]).wait()
        plsc.subcore_barrier()

        off = 0
        while off < rps:
            n = min(B, rps - off)
            pltpu.sync_copy(acc.at[pl.ds(sid * rps + off, n)],
                            out_h.at[cid, pl.ds(sid * rps + off, n)])
            off += n

    return k(xn, src, dst, et, recip)


def kernel(x, edge_src0, edge_dst0, etype0, edge_src1, edge_dst1, etype1,
           n_dst0, n_dst1, Wself, Wneigh, bconv, Wskip, bskip, gamma, beta,
           W1, b1, g1, be1, W2, b2):
    wn0 = jnp.transpose(Wneigh[0], (1, 0, 2)).reshape(D, NE * D)
    wc0 = Wskip[0] + Wself[0].sum(0)
    bc0 = bskip[0] + bconv[0].sum(0)
    wn1 = jnp.transpose(Wneigh[1], (1, 0, 2)).reshape(D, NE * D)
    wc1 = Wskip[1] + Wself[1].sum(0)
    bc1 = bskip[1] + bconv[1].sum(0)
    dout = W2.shape[1]
    w2p = jnp.pad(W2, ((0, 0), (0, 256 - dout)))
    b2p = jnp.pad(b2, (0, 256 - dout))

    recip0, recip1 = _sc_counts(edge_dst0, etype0, edge_dst1, etype1)

    y0a, y0b = _tc_project(x, wn0, wc0, bc0, br=1000)
    xn0 = y0a.reshape(x.shape[0] * NE, D)
    hp0 = _sc_agg(xn0, edge_src0, edge_dst0, etype0, recip0, N0P)

    y1a, y1b = _tc_mid(y0b, hp0, gamma[0], beta[0], wn1, wc1, bc1, N0)
    xn1 = y1a.reshape(N0 * NE, D)
    hp1 = _sc_agg(xn1, edge_src1, edge_dst1, etype1, recip1, N1)

    out = _tc_final(y1b, hp1, gamma[1], beta[1], W1, b1, g1, be1, w2p, b2p)
    return out[:, :dout]
